# baseline (speedup 1.0000x reference)
import jax
import jax.numpy as jnp
from jax import lax
from jax.experimental import pallas as pl
from jax.experimental.pallas import tpu as pltpu
from jax.experimental.pallas import tpu_sc as plsc

_BATCH = 4096
_MAXLEN = 200
_EMBED = 64
_VOCAB = 1000000
_NW = 32
_SEQS_W = _BATCH // _NW
_PAIR2 = _SEQS_W // 4
_ROWS_P = 2 * _MAXLEN
_NVEC = _ROWS_P // 16
_Q = ((0, 0, 104), (0, 104, 96), (1, 0, 104), (1, 104, 96))


def _body(xf_hbm, tok_hbm, pos_hbm, out_hbm,
          pos_v, idxr0, idxr1, idxg0, idxg1,
          wide0, wide1, outb0, outb1,
          gsem0, gsem1, wsem0, wsem1):
    wid = lax.axis_index("s") * 2 + lax.axis_index("c")
    pair_base = wid * (_SEQS_W // 2)

    idxr = (idxr0, idxr1)
    idxg = (idxg0, idxg1)
    wide = (wide0, wide1)
    outb = (outb0, outb1)
    gsem = (gsem0, gsem1)
    wsem = (wsem0, wsem1)

    pltpu.sync_copy(pos_hbm, pos_v)

    def stage_pair(pair, pb):
        base = (pair_base + pair) * _ROWS_P
        pltpu.sync_copy(
            xf_hbm.at[pl.ds(base, _ROWS_P)], idxr[pb].at[pl.ds(0, _ROWS_P)]
        )

        def sh(v, _):
            sl = pl.ds(v * 16, 16)
            idxg[pb][sl] = lax.shift_right_logical(idxr[pb][sl], 1)
            return 0

        lax.fori_loop(0, _NVEC, sh, 0)

    def fire_gather(pb, q):
        s2, off, ln = _Q[q]
        qb = q % 2
        pltpu.async_copy(
            tok_hbm.at[idxg[pb].at[pl.ds(s2 * _MAXLEN + off, ln)]],
            wide[qb],
            gsem[qb],
        )

    def wait_gather(pb, q):
        s2, off, ln = _Q[q]
        qb = q % 2
        pltpu.make_async_copy(
            tok_hbm.at[idxg[pb].at[pl.ds(s2 * _MAXLEN + off, ln)]],
            wide[qb],
            gsem[qb],
        ).wait()

    def repack(pair, pb, q):
        s2, off, ln = _Q[q]
        qb = q % 2

        def rep_body(i, _):
            row = s2 * _MAXLEN + off + i
            xv = idxr[pb][pl.ds(row, 16)][0]
            o = (xv & 1) * _EMBED
            p = off + i
            for j in range(4):
                sl = pl.ds(16 * j, 16)
                outb[qb][i, sl] = wide[qb][i, pl.ds(o + 16 * j, 16)] \
                    + pos_v[p, sl]
            return 0

        lax.fori_loop(0, ln, rep_body, 0)

    def seq_abs(pair, q):
        s2 = _Q[q][0]
        return (pair_base + pair) * 2 + s2

    def fire_write(pair, q):
        _, off, ln = _Q[q]
        qb = q % 2
        pltpu.async_copy(
            outb[qb], out_hbm.at[seq_abs(pair, q), pl.ds(off, ln)], wsem[qb]
        )

    def wait_write(pair, q):
        _, off, ln = _Q[q]
        qb = q % 2
        pltpu.make_async_copy(
            outb[qb], out_hbm.at[seq_abs(pair, q), pl.ds(off, ln)], wsem[qb]
        ).wait()

    def do_quarter(pair, pb, q, first, last):
        wait_gather(pb, q)
        if q < 3:
            if q == 2 and last is not None:
                @pl.when(jnp.logical_not(last))
                def _():
                    stage_pair(pair + 1, pb ^ 1)
            elif q == 2:
                stage_pair(pair + 1, pb ^ 1)
            fire_gather(pb, q + 1)
        else:
            if last is None:
                fire_gather(pb ^ 1, 0)
            else:
                @pl.when(jnp.logical_not(last))
                def _():
                    fire_gather(pb ^ 1, 0)
        if first is None:
            wait_write(pair, q - 2) if q >= 2 else wait_write(pair - 1, q + 2)
        elif q >= 2:
            wait_write(pair, q - 2)
        else:
            @pl.when(jnp.logical_not(first))
            def _():
                wait_write(pair - 1, q + 2)
        repack(pair, pb, q)
        fire_write(pair, q)

    stage_pair(0, 0)
    fire_gather(0, 0)

    def pair2_body(p2, _):
        for pb in (0, 1):
            pair = p2 * 2 + pb
            first = (pair == 0) if pb == 0 else None
            last = (pair == _SEQS_W // 2 - 1) if pb == 1 else None
            for q in range(4):
                do_quarter(pair, pb, q, first, last)
        return 0

    lax.fori_loop(0, _PAIR2, pair2_body, 0)

    wait_write(_SEQS_W // 2 - 1, 2)
    wait_write(_SEQS_W // 2 - 1, 3)


_mesh = plsc.VectorSubcoreMesh(core_axis_name="c", subcore_axis_name="s")

_embed = pl.kernel(
    _body,
    out_type=jax.ShapeDtypeStruct((_BATCH, _MAXLEN, _EMBED), jnp.float32),
    mesh=_mesh,
    scratch_types=[
        pltpu.VMEM((_MAXLEN, _EMBED), jnp.float32),
        pltpu.VMEM((_ROWS_P + 16,), jnp.int32),
        pltpu.VMEM((_ROWS_P + 16,), jnp.int32),
        pltpu.VMEM((_ROWS_P,), jnp.int32),
        pltpu.VMEM((_ROWS_P,), jnp.int32),
        pltpu.VMEM((104, 2 * _EMBED), jnp.float32),
        pltpu.VMEM((96, 2 * _EMBED), jnp.float32),
        pltpu.VMEM((104, _EMBED), jnp.float32),
        pltpu.VMEM((96, _EMBED), jnp.float32),
        pltpu.SemaphoreType.DMA,
        pltpu.SemaphoreType.DMA,
        pltpu.SemaphoreType.DMA,
        pltpu.SemaphoreType.DMA,
    ],
    compiler_params=pltpu.CompilerParams(use_tc_tiling_on_sc=True),
)


@jax.jit
def kernel(x, token_table, pos_table):
    xf = x.astype(jnp.int32).reshape(-1)
    tok2 = token_table.reshape(_VOCAB // 2, 2 * _EMBED)
    return _embed(xf, tok2, pos_table)


# speedup vs baseline: 1.3226x; 1.3226x over previous
import jax
import jax.numpy as jnp
from jax import lax
from jax.experimental import pallas as pl
from jax.experimental.pallas import tpu as pltpu
from jax.experimental.pallas import tpu_sc as plsc

_BATCH = 4096
_MAXLEN = 200
_EMBED = 64
_VOCAB = 1000000
_NW = 32
_SEQS_W = _BATCH // _NW
_CH = ((0, 112, 7), (112, 88, 6))


def _body(x_hbm, tok_hbm, pos_hbm, out_hbm,
          xall, pos_v, wide0, wide1, outb0, outb1,
          gsem0, gsem1, wsem0, wsem1):
    wid = lax.axis_index("s") * 2 + lax.axis_index("c")
    seq_base = wid * _SEQS_W

    wide = (wide0, wide1)
    outb = (outb0, outb1)
    gsem = (gsem0, gsem1)
    wsem = (wsem0, wsem1)

    pltpu.sync_copy(x_hbm.at[pl.ds(seq_base, _SEQS_W)], xall)
    pltpu.sync_copy(pos_hbm, pos_v)

    def fire_gather(seq, h):
        off, ln, nv = _CH[h]
        for k in range(nv):
            start = min(off + 16 * k, _MAXLEN - 16)
            xv = xall[seq, pl.ds(start, 16)]
            line = lax.shift_right_logical(xv, 1)
            pltpu.async_copy(
                tok_hbm.at[line], wide[h].at[pl.ds(16 * k, 16)], gsem[h]
            )

    def wait_gather(h):
        off, ln, nv = _CH[h]
        zero = jnp.zeros((16,), jnp.int32)
        for k in range(nv):
            pltpu.make_async_copy(
                tok_hbm.at[zero], wide[h].at[pl.ds(16 * k, 16)], gsem[h]
            ).wait()

    def repack(seq, h):
        off, ln, nv = _CH[h]
        full_groups = ln // 16
        tail = ln - 16 * full_groups

        def do_rows(xv_start, lane0, wide_base, outb_base, pos_base, nrows):
            xv = xall[seq, pl.ds(xv_start, 16)]
            par = xv & 1
            for r in range(nrows):
                pf = jax.lax.convert_element_type(
                    par.at[jnp.full((16,), lane0 + r, jnp.int32)].get(
                        mode="promise_in_bounds"), jnp.float32)
                wr = wide_base + r
                orow = outb_base + r
                p = pos_base + r
                for j in range(4):
                    sl = pl.ds(16 * j, 16)
                    vl = wide[h][wr, sl]
                    vr = wide[h][wr, pl.ds(64 + 16 * j, 16)]
                    outb[h][orow, sl] = (
                        vl + pf * (vr - vl) + pos_v[p, sl])

        def grp_body(g, _):
            base = 16 * g
            do_rows(off + base, 0, base, base, off + base, 16)
            return 0

        lax.fori_loop(0, full_groups, grp_body, 0)
        if tail:
            do_rows(_MAXLEN - 16, 16 - tail, 16 * nv - tail,
                    16 * full_groups, _MAXLEN - tail, tail)

    def fire_write(seq, h):
        off, ln, nv = _CH[h]
        pltpu.async_copy(
            outb[h].at[pl.ds(0, ln)],
            out_hbm.at[seq_base + seq, pl.ds(off, ln)],
            wsem[h],
        )

    def wait_write(seq, h):
        off, ln, nv = _CH[h]
        pltpu.make_async_copy(
            outb[h].at[pl.ds(0, ln)],
            out_hbm.at[seq_base + seq, pl.ds(off, ln)],
            wsem[h],
        ).wait()

    fire_gather(0, 0)

    def seq_body(seq, _):
        wait_gather(0)
        fire_gather(seq, 1)

        @pl.when(seq > 0)
        def _():
            wait_write(seq - 1, 0)

        repack(seq, 0)
        fire_write(seq, 0)

        wait_gather(1)

        @pl.when(seq < _SEQS_W - 1)
        def _():
            fire_gather(seq + 1, 0)

        @pl.when(seq > 0)
        def _():
            wait_write(seq - 1, 1)

        repack(seq, 1)
        fire_write(seq, 1)
        return 0

    lax.fori_loop(0, _SEQS_W, seq_body, 0)

    wait_write(_SEQS_W - 1, 0)
    wait_write(_SEQS_W - 1, 1)


_mesh = plsc.VectorSubcoreMesh(core_axis_name="c", subcore_axis_name="s")

_embed = pl.kernel(
    _body,
    out_type=jax.ShapeDtypeStruct((_BATCH, _MAXLEN, _EMBED), jnp.float32),
    mesh=_mesh,
    scratch_types=[
        pltpu.VMEM((_SEQS_W, _MAXLEN), jnp.int32),
        pltpu.VMEM((_MAXLEN, _EMBED), jnp.float32),
        pltpu.VMEM((112, 2 * _EMBED), jnp.float32),
        pltpu.VMEM((96, 2 * _EMBED), jnp.float32),
        pltpu.VMEM((112, _EMBED), jnp.float32),
        pltpu.VMEM((96, _EMBED), jnp.float32),
        pltpu.SemaphoreType.DMA,
        pltpu.SemaphoreType.DMA,
        pltpu.SemaphoreType.DMA,
        pltpu.SemaphoreType.DMA,
    ],
    compiler_params=pltpu.CompilerParams(use_tc_tiling_on_sc=True),
)


@jax.jit
def kernel(x, token_table, pos_table):
    tok2 = token_table.reshape(_VOCAB // 2, 2 * _EMBED)
    return _embed(x.astype(jnp.int32), tok2, pos_table)
